# Initial kernel scaffold; baseline (speedup 1.0000x reference)
import jax
import jax.numpy as jnp
from jax.experimental import pallas as pl


def kernel(x, edge_index, W_e1, b_e1, W_e2, b_e2, W_n1, b_n1, W_n2, b_n2):
    raise NotImplementedError("write your pallas kernel here")


# speedup vs baseline: 3.5447x; 3.5447x over previous
import functools

import jax
import jax.numpy as jnp
from jax import lax
from jax.experimental import pallas as pl
from jax.experimental.pallas import tpu as pltpu
from jax.experimental.pallas import tpu_sc as plsc

NC = 2
NS = 16
NW = NC * NS
CH = 80


def _dot_t(a, b):
    return lax.dot_general(a, b, (((1,), (1,)), ((), ())),
                           preferred_element_type=jnp.float32)


def _uv_body(x_ref, w_ref, b_ref, u_ref, v_ref):
    d = x_ref.shape[1]
    xv = x_ref[...]
    u_ref[...] = _dot_t(xv, w_ref[:, :d]) + b_ref[...]
    v_ref[...] = _dot_t(xv, w_ref[:, d:])


def _edge_body(s_ref, w_ref, b_ref, m_ref):
    sv = jnp.maximum(s_ref[...], 0.0)
    m_ref[...] = jnp.maximum(_dot_t(sv, w_ref[...]) + b_ref[...], 0.0)


def _node_body(x_ref, p_ref, w1x_ref, w1a_ref, b1_ref, w2_ref, b2_ref, h_ref):
    xv = x_ref[...]
    agg = p_ref[0] + p_ref[1]
    h1 = jnp.maximum(_dot_t(xv, w1x_ref[...]) + _dot_t(agg, w1a_ref[...])
                     + b1_ref[...], 0.0)
    h_ref[...] = _dot_t(h1, w2_ref[...]) + b2_ref[...] + xv


def _gather_add_body(u_hbm, v_hbm, row_hbm, col_hbm, out_hbm,
                     ridx, cidx, ubuf, vbuf, gsem1, gsem2):
    epw = ridx.shape[0]
    nch = epw // CH
    h = ubuf.shape[1]
    wid = lax.axis_index("s") * NC + lax.axis_index("c")
    ebase = wid * epw
    pltpu.sync_copy(row_hbm.at[pl.ds(ebase, epw)], ridx)
    pltpu.sync_copy(col_hbm.at[pl.ds(ebase, epw)], cidx)

    @pl.loop(0, nch)
    def _chunk(i):
        off = i * CH
        cp1 = pltpu.async_copy(u_hbm.at[ridx.at[pl.ds(off, CH)]], ubuf, gsem1)
        cp2 = pltpu.async_copy(v_hbm.at[cidx.at[pl.ds(off, CH)]], vbuf, gsem2)
        cp1.wait()
        cp2.wait()

        @pl.loop(0, CH)
        def _row(r):
            for j in range(h // 16):
                sl = pl.ds(j * 16, 16)
                ubuf[r, sl] = ubuf[r, sl] + vbuf[r, sl]

        pltpu.sync_copy(ubuf, out_hbm.at[pl.ds(ebase + off, CH)])


def _scatter_add_body(m_hbm, row_hbm, zeros_hbm, out_hbm, rbuf, mbuf, agg_sh):
    n = agg_sh.shape[0]
    epw = m_hbm.shape[0] // NW
    nch = epw // CH
    rpt = (n // NS) & ~7
    tail = n - NS * rpt
    c = lax.axis_index("c")
    sub = lax.axis_index("s")
    wid = sub * NC + c
    ebase = wid * epw
    nb = sub * rpt
    pltpu.sync_copy(zeros_hbm.at[pl.ds(nb, rpt)], agg_sh.at[pl.ds(nb, rpt)])
    if tail:
        @pl.when(sub == NS - 1)
        def _init_tail():
            pltpu.sync_copy(zeros_hbm.at[pl.ds(NS * rpt, tail)],
                            agg_sh.at[pl.ds(NS * rpt, tail)])
    plsc.subcore_barrier()

    @pl.loop(0, nch)
    def _chunk(i):
        off = ebase + i * CH
        pltpu.sync_copy(row_hbm.at[pl.ds(off, CH)], rbuf)
        pltpu.sync_copy(m_hbm.at[pl.ds(off, CH)], mbuf)
        pltpu.sync_copy(mbuf, agg_sh.at[rbuf], add=True)

    plsc.subcore_barrier()
    pltpu.sync_copy(agg_sh.at[pl.ds(nb, rpt)], out_hbm.at[c, pl.ds(nb, rpt)])
    if tail:
        @pl.when(sub == NS - 1)
        def _out_tail():
            pltpu.sync_copy(agg_sh.at[pl.ds(NS * rpt, tail)],
                            out_hbm.at[c, pl.ds(NS * rpt, tail)])


def kernel(x, edge_index, W_e1, b_e1, W_e2, b_e2, W_n1, b_n1, W_n2, b_n2):
    n, d = x.shape
    e = edge_index.shape[1]
    h = W_e1.shape[0]
    assert e % (NW * CH) == 0 and n % NS == 0

    row = edge_index[0].astype(jnp.int32)
    col = edge_index[1].astype(jnp.int32)

    u, v = pl.pallas_call(
        _uv_body,
        out_shape=(jax.ShapeDtypeStruct((n, h), jnp.float32),
                   jax.ShapeDtypeStruct((n, h), jnp.float32)),
    )(x, W_e1, b_e1.reshape(1, h))

    mesh = plsc.VectorSubcoreMesh(core_axis_name="c", subcore_axis_name="s",
                                  num_cores=NC, num_subcores=NS)
    epw = e // NW
    s = pl.kernel(
        _gather_add_body,
        out_type=jax.ShapeDtypeStruct((e, h), jnp.float32),
        mesh=mesh,
        scratch_types=[
            pltpu.VMEM((epw,), jnp.int32),
            pltpu.VMEM((epw,), jnp.int32),
            pltpu.VMEM((CH, h), jnp.float32),
            pltpu.VMEM((CH, h), jnp.float32),
            pltpu.SemaphoreType.DMA,
            pltpu.SemaphoreType.DMA,
        ],
    )(u, v, row, col)

    eb = 2000
    m = pl.pallas_call(
        _edge_body,
        grid=(e // eb,),
        in_specs=[
            pl.BlockSpec((eb, h), lambda i: (i, 0)),
            pl.BlockSpec((h, h), lambda i: (0, 0)),
            pl.BlockSpec((1, h), lambda i: (0, 0)),
        ],
        out_specs=pl.BlockSpec((eb, h), lambda i: (i, 0)),
        out_shape=jax.ShapeDtypeStruct((e, h), jnp.float32),
    )(s, W_e2, b_e2.reshape(1, h))

    zeros = jnp.zeros((n, h), jnp.float32)
    partials = pl.kernel(
        _scatter_add_body,
        out_type=jax.ShapeDtypeStruct((NC, n, h), jnp.float32),
        mesh=mesh,
        scratch_types=[
            pltpu.VMEM((CH,), jnp.int32),
            pltpu.VMEM((CH, h), jnp.float32),
            pltpu.VMEM_SHARED((n, h), jnp.float32),
        ],
    )(m, row, zeros)

    nb = 2000
    hout = pl.pallas_call(
        _node_body,
        grid=(n // nb,),
        in_specs=[
            pl.BlockSpec((nb, d), lambda i: (i, 0)),
            pl.BlockSpec((NC, nb, h), lambda i: (0, i, 0)),
            pl.BlockSpec((h, d), lambda i: (0, 0)),
            pl.BlockSpec((h, h), lambda i: (0, 0)),
            pl.BlockSpec((1, h), lambda i: (0, 0)),
            pl.BlockSpec((d, h), lambda i: (0, 0)),
            pl.BlockSpec((1, d), lambda i: (0, 0)),
        ],
        out_specs=pl.BlockSpec((nb, d), lambda i: (i, 0)),
        out_shape=jax.ShapeDtypeStruct((n, d), jnp.float32),
    )(x, partials, W_n1[:, :d], W_n1[:, d:], b_n1.reshape(1, h),
      W_n2, b_n2.reshape(1, d))

    return (hout, m)


# speedup vs baseline: 3.8118x; 1.0753x over previous
import functools

import jax
import jax.numpy as jnp
from jax import lax
from jax.experimental import pallas as pl
from jax.experimental.pallas import tpu as pltpu
from jax.experimental.pallas import tpu_sc as plsc

NC = 2
NS = 16
NW = NC * NS
CH = 80


def _dot_t(a, b):
    return lax.dot_general(a, b, (((1,), (1,)), ((), ())),
                           preferred_element_type=jnp.float32)


def _uv_body(x_ref, w_ref, b_ref, u_ref, v_ref):
    d = x_ref.shape[1]
    xv = x_ref[...]
    u_ref[...] = _dot_t(xv, w_ref[:, :d]) + b_ref[...]
    v_ref[...] = _dot_t(xv, w_ref[:, d:])


def _edge_body(s_ref, w_ref, b_ref, m_ref):
    sv = jnp.maximum(s_ref[...], 0.0)
    m_ref[...] = jnp.maximum(_dot_t(sv, w_ref[...]) + b_ref[...], 0.0)


def _node_body(x_ref, p_ref, w1x_ref, w1a_ref, b1_ref, w2_ref, b2_ref, h_ref):
    xv = x_ref[...]
    agg = p_ref[0] + p_ref[1]
    h1 = jnp.maximum(_dot_t(xv, w1x_ref[...]) + _dot_t(agg, w1a_ref[...])
                     + b1_ref[...], 0.0)
    h_ref[...] = _dot_t(h1, w2_ref[...]) + b2_ref[...] + xv


def _gather_add_body(u_hbm, v_hbm, row_hbm, col_hbm, out_hbm,
                     ridx, cidx, ub0, vb0, ob0, ub1, vb1, ob1,
                     gs0, gs1, ws0, ws1):
    epw = ridx.shape[0]
    nch = epw // CH
    h = ub0.shape[1]
    wid = lax.axis_index("s") * NC + lax.axis_index("c")
    ebase = wid * epw
    pltpu.sync_copy(row_hbm.at[pl.ds(ebase, epw)], ridx)
    pltpu.sync_copy(col_hbm.at[pl.ds(ebase, epw)], cidx)

    bufs = ((ub0, vb0, ob0, gs0, ws0), (ub1, vb1, ob1, gs1, ws1))

    def fire(i, ub, vb, gs):
        off = i * CH
        pltpu.async_copy(u_hbm.at[ridx.at[pl.ds(off, CH)]], ub, gs)
        pltpu.async_copy(v_hbm.at[cidx.at[pl.ds(off, CH)]], vb, gs)

    def wait_gather(ub, vb, gs):
        pltpu.make_async_copy(u_hbm.at[ridx.at[pl.ds(0, CH)]], ub, gs).wait()
        pltpu.make_async_copy(v_hbm.at[cidx.at[pl.ds(0, CH)]], vb, gs).wait()

    def compute(ub, vb, ob):
        @pl.loop(0, CH, unroll=4)
        def _row(r):
            for j in range(h // 16):
                sl = pl.ds(j * 16, 16)
                ob[r, sl] = ub[r, sl] + vb[r, sl]

    def fire_wb(i, ob, ws):
        pltpu.async_copy(ob, out_hbm.at[pl.ds(ebase + i * CH, CH)], ws)

    def wait_wb(ob, ws):
        pltpu.make_async_copy(ob, out_hbm.at[pl.ds(ebase, CH)], ws).wait()

    fire(0, ub0, vb0, gs0)
    fire(1, ub1, vb1, gs1)

    @pl.loop(0, nch // 2)
    def _main(k):
        for b in range(2):
            ub, vb, ob, gs, ws = bufs[b]
            i = 2 * k + b
            wait_gather(ub, vb, gs)

            @pl.when(k > 0)
            def _():
                wait_wb(ob, ws)

            compute(ub, vb, ob)
            fire_wb(i, ob, ws)

            @pl.when(i + 2 < nch)
            def _():
                fire(i + 2, ub, vb, gs)

    wait_gather(ub0, vb0, gs0)
    wait_wb(ob0, ws0)
    compute(ub0, vb0, ob0)
    fire_wb(nch - 1, ob0, ws0)
    wait_wb(ob0, ws0)
    wait_wb(ob1, ws1)


def _scatter_add_body(m_hbm, row_hbm, zeros_hbm, out_hbm,
                      rb0, mb0, rb1, mb1, agg_sh, ds0, ds1):
    n = agg_sh.shape[0]
    epw = m_hbm.shape[0] // NW
    nch = epw // CH
    rpt = (n // NS) & ~7
    tail = n - NS * rpt
    c = lax.axis_index("c")
    sub = lax.axis_index("s")
    wid = sub * NC + c
    ebase = wid * epw
    nb = sub * rpt
    pltpu.sync_copy(zeros_hbm.at[pl.ds(nb, rpt)], agg_sh.at[pl.ds(nb, rpt)])
    if tail:
        @pl.when(sub == NS - 1)
        def _init_tail():
            pltpu.sync_copy(zeros_hbm.at[pl.ds(NS * rpt, tail)],
                            agg_sh.at[pl.ds(NS * rpt, tail)])
    plsc.subcore_barrier()

    bufs = ((rb0, mb0, ds0), (rb1, mb1, ds1))

    def fire(i, rb, mb, dsm):
        off = ebase + i * CH
        pltpu.async_copy(row_hbm.at[pl.ds(off, CH)], rb, dsm)
        pltpu.async_copy(m_hbm.at[pl.ds(off, CH)], mb, dsm)

    def wait_fire(rb, mb, dsm):
        pltpu.make_async_copy(row_hbm.at[pl.ds(ebase, CH)], rb, dsm).wait()
        pltpu.make_async_copy(m_hbm.at[pl.ds(ebase, CH)], mb, dsm).wait()

    fire(0, rb0, mb0, ds0)
    fire(1, rb1, mb1, ds1)

    @pl.loop(0, nch // 2)
    def _main(k):
        for b in range(2):
            rb, mb, dsm = bufs[b]
            i = 2 * k + b
            wait_fire(rb, mb, dsm)
            pltpu.sync_copy(mb, agg_sh.at[rb], add=True)

            @pl.when(i + 2 < nch)
            def _():
                fire(i + 2, rb, mb, dsm)

    wait_fire(rb0, mb0, ds0)
    pltpu.sync_copy(mb0, agg_sh.at[rb0], add=True)

    plsc.subcore_barrier()
    pltpu.sync_copy(agg_sh.at[pl.ds(nb, rpt)], out_hbm.at[c, pl.ds(nb, rpt)])
    if tail:
        @pl.when(sub == NS - 1)
        def _out_tail():
            pltpu.sync_copy(agg_sh.at[pl.ds(NS * rpt, tail)],
                            out_hbm.at[c, pl.ds(NS * rpt, tail)])


def kernel(x, edge_index, W_e1, b_e1, W_e2, b_e2, W_n1, b_n1, W_n2, b_n2):
    n, d = x.shape
    e = edge_index.shape[1]
    h = W_e1.shape[0]
    assert e % (NW * CH) == 0 and n % NS == 0

    row = edge_index[0].astype(jnp.int32)
    col = edge_index[1].astype(jnp.int32)

    u, v = pl.pallas_call(
        _uv_body,
        out_shape=(jax.ShapeDtypeStruct((n, h), jnp.float32),
                   jax.ShapeDtypeStruct((n, h), jnp.float32)),
    )(x, W_e1, b_e1.reshape(1, h))

    mesh = plsc.VectorSubcoreMesh(core_axis_name="c", subcore_axis_name="s",
                                  num_cores=NC, num_subcores=NS)
    epw = e // NW
    s = pl.kernel(
        _gather_add_body,
        out_type=jax.ShapeDtypeStruct((e, h), jnp.float32),
        mesh=mesh,
        scratch_types=[
            pltpu.VMEM((epw,), jnp.int32),
            pltpu.VMEM((epw,), jnp.int32),
            pltpu.VMEM((CH, h), jnp.float32),
            pltpu.VMEM((CH, h), jnp.float32),
            pltpu.VMEM((CH, h), jnp.float32),
            pltpu.VMEM((CH, h), jnp.float32),
            pltpu.VMEM((CH, h), jnp.float32),
            pltpu.VMEM((CH, h), jnp.float32),
            pltpu.SemaphoreType.DMA,
            pltpu.SemaphoreType.DMA,
            pltpu.SemaphoreType.DMA,
            pltpu.SemaphoreType.DMA,
        ],
    )(u, v, row, col)

    eb = 2000
    m = pl.pallas_call(
        _edge_body,
        grid=(e // eb,),
        in_specs=[
            pl.BlockSpec((eb, h), lambda i: (i, 0)),
            pl.BlockSpec((h, h), lambda i: (0, 0)),
            pl.BlockSpec((1, h), lambda i: (0, 0)),
        ],
        out_specs=pl.BlockSpec((eb, h), lambda i: (i, 0)),
        out_shape=jax.ShapeDtypeStruct((e, h), jnp.float32),
    )(s, W_e2, b_e2.reshape(1, h))

    zeros = jnp.zeros((n, h), jnp.float32)
    partials = pl.kernel(
        _scatter_add_body,
        out_type=jax.ShapeDtypeStruct((NC, n, h), jnp.float32),
        mesh=mesh,
        scratch_types=[
            pltpu.VMEM((CH,), jnp.int32),
            pltpu.VMEM((CH, h), jnp.float32),
            pltpu.VMEM((CH,), jnp.int32),
            pltpu.VMEM((CH, h), jnp.float32),
            pltpu.VMEM_SHARED((n, h), jnp.float32),
            pltpu.SemaphoreType.DMA,
            pltpu.SemaphoreType.DMA,
        ],
    )(m, row, zeros)

    nb = 2000
    hout = pl.pallas_call(
        _node_body,
        grid=(n // nb,),
        in_specs=[
            pl.BlockSpec((nb, d), lambda i: (i, 0)),
            pl.BlockSpec((NC, nb, h), lambda i: (0, i, 0)),
            pl.BlockSpec((h, d), lambda i: (0, 0)),
            pl.BlockSpec((h, h), lambda i: (0, 0)),
            pl.BlockSpec((1, h), lambda i: (0, 0)),
            pl.BlockSpec((d, h), lambda i: (0, 0)),
            pl.BlockSpec((1, d), lambda i: (0, 0)),
        ],
        out_specs=pl.BlockSpec((nb, d), lambda i: (i, 0)),
        out_shape=jax.ShapeDtypeStruct((n, d), jnp.float32),
    )(x, partials, W_n1[:, :d], W_n1[:, d:], b_n1.reshape(1, h),
      W_n2, b_n2.reshape(1, d))

    return (hout, m)


# speedup vs baseline: 5.4613x; 1.4327x over previous
import functools

import jax
import jax.numpy as jnp
from jax import lax
from jax.experimental import pallas as pl
from jax.experimental.pallas import tpu as pltpu
from jax.experimental.pallas import tpu_sc as plsc

NC = 2
NS = 16
NW = NC * NS
CH = 80


def _dot_t(a, b):
    return lax.dot_general(a, b, (((1,), (1,)), ((), ())),
                           preferred_element_type=jnp.float32)


def _uv_body(x_ref, w_ref, b_ref, u_ref, v_ref):
    d = x_ref.shape[1]
    xv = x_ref[...]
    u_ref[...] = _dot_t(xv, w_ref[:, :d]) + b_ref[...]
    v_ref[...] = _dot_t(xv, w_ref[:, d:])


def _edge_body(s_ref, w_ref, b_ref, m_ref):
    sv = jnp.maximum(s_ref[...], 0.0)
    m_ref[...] = jnp.maximum(_dot_t(sv, w_ref[...]) + b_ref[...], 0.0)


def _node_body(x_ref, p_ref, w1x_ref, w1a_ref, b1_ref, w2_ref, b2_ref, h_ref):
    xv = x_ref[...]
    agg = p_ref[0] + p_ref[1]
    h1 = jnp.maximum(_dot_t(xv, w1x_ref[...]) + _dot_t(agg, w1a_ref[...])
                     + b1_ref[...], 0.0)
    h_ref[...] = _dot_t(h1, w2_ref[...]) + b2_ref[...] + xv


def _gather_add_body(u_hbm, v_hbm, row_hbm, col_hbm, out_hbm,
                     ridx, cidx, ub0, vb0, ob0, ub1, vb1, ob1,
                     gs0, gs1, ws0, ws1):
    epw = ridx.shape[0]
    nch = epw // CH
    h = ub0.shape[1]
    wid = lax.axis_index("s") * NC + lax.axis_index("c")
    ebase = wid * epw
    pltpu.sync_copy(row_hbm.at[pl.ds(ebase, epw)], ridx)
    pltpu.sync_copy(col_hbm.at[pl.ds(ebase, epw)], cidx)

    bufs = ((ub0, vb0, ob0, gs0, ws0), (ub1, vb1, ob1, gs1, ws1))

    def fire(i, ub, vb, gs):
        off = i * CH
        pltpu.async_copy(u_hbm.at[ridx.at[pl.ds(off, CH)]], ub, gs)
        pltpu.async_copy(v_hbm.at[cidx.at[pl.ds(off, CH)]], vb, gs)

    def wait_gather(ub, vb, gs):
        pltpu.make_async_copy(u_hbm.at[ridx.at[pl.ds(0, CH)]], ub, gs).wait()
        pltpu.make_async_copy(v_hbm.at[cidx.at[pl.ds(0, CH)]], vb, gs).wait()

    def compute(ub, vb, ob):
        @plsc.parallel_loop(0, CH)
        def _row(r):
            for j in range(h // 16):
                sl = pl.ds(j * 16, 16)
                ob[r, sl] = ub[r, sl] + vb[r, sl]

    def fire_wb(i, ob, ws):
        pltpu.async_copy(ob, out_hbm.at[pl.ds(ebase + i * CH, CH)], ws)

    def wait_wb(ob, ws):
        pltpu.make_async_copy(ob, out_hbm.at[pl.ds(ebase, CH)], ws).wait()

    fire(0, ub0, vb0, gs0)
    fire(1, ub1, vb1, gs1)

    @pl.loop(0, nch // 2)
    def _main(k):
        for b in range(2):
            ub, vb, ob, gs, ws = bufs[b]
            i = 2 * k + b
            wait_gather(ub, vb, gs)

            @pl.when(k > 0)
            def _():
                wait_wb(ob, ws)

            compute(ub, vb, ob)
            fire_wb(i, ob, ws)

            @pl.when(i + 2 < nch)
            def _():
                fire(i + 2, ub, vb, gs)

    wait_gather(ub0, vb0, gs0)
    wait_wb(ob0, ws0)
    compute(ub0, vb0, ob0)
    fire_wb(nch - 1, ob0, ws0)
    wait_wb(ob0, ws0)
    wait_wb(ob1, ws1)


def _scatter_add_body(m_hbm, row_hbm, zeros_hbm, out_hbm,
                      rb0, mb0, rb1, mb1, agg_sh, ds0, ds1):
    n = agg_sh.shape[0]
    epw = m_hbm.shape[0] // NW
    nch = epw // CH
    rpt = (n // NS) & ~7
    tail = n - NS * rpt
    c = lax.axis_index("c")
    sub = lax.axis_index("s")
    wid = sub * NC + c
    ebase = wid * epw
    nb = sub * rpt
    pltpu.sync_copy(zeros_hbm.at[pl.ds(nb, rpt)], agg_sh.at[pl.ds(nb, rpt)])
    if tail:
        @pl.when(sub == NS - 1)
        def _init_tail():
            pltpu.sync_copy(zeros_hbm.at[pl.ds(NS * rpt, tail)],
                            agg_sh.at[pl.ds(NS * rpt, tail)])
    plsc.subcore_barrier()

    bufs = ((rb0, mb0, ds0), (rb1, mb1, ds1))

    def fire(i, rb, mb, dsm):
        off = ebase + i * CH
        pltpu.async_copy(row_hbm.at[pl.ds(off, CH)], rb, dsm)
        pltpu.async_copy(m_hbm.at[pl.ds(off, CH)], mb, dsm)

    def wait_fire(rb, mb, dsm):
        pltpu.make_async_copy(row_hbm.at[pl.ds(ebase, CH)], rb, dsm).wait()
        pltpu.make_async_copy(m_hbm.at[pl.ds(ebase, CH)], mb, dsm).wait()

    fire(0, rb0, mb0, ds0)
    fire(1, rb1, mb1, ds1)

    @pl.loop(0, nch // 2)
    def _main(k):
        for b in range(2):
            rb, mb, dsm = bufs[b]
            i = 2 * k + b
            wait_fire(rb, mb, dsm)
            pltpu.sync_copy(mb, agg_sh.at[rb], add=True)

            @pl.when(i + 2 < nch)
            def _():
                fire(i + 2, rb, mb, dsm)

    wait_fire(rb0, mb0, ds0)
    pltpu.sync_copy(mb0, agg_sh.at[rb0], add=True)

    plsc.subcore_barrier()
    pltpu.sync_copy(agg_sh.at[pl.ds(nb, rpt)], out_hbm.at[c, pl.ds(nb, rpt)])
    if tail:
        @pl.when(sub == NS - 1)
        def _out_tail():
            pltpu.sync_copy(agg_sh.at[pl.ds(NS * rpt, tail)],
                            out_hbm.at[c, pl.ds(NS * rpt, tail)])


def kernel(x, edge_index, W_e1, b_e1, W_e2, b_e2, W_n1, b_n1, W_n2, b_n2):
    n, d = x.shape
    e = edge_index.shape[1]
    h = W_e1.shape[0]
    assert e % (NW * CH) == 0 and n % NS == 0

    row = edge_index[0].astype(jnp.int32)
    col = edge_index[1].astype(jnp.int32)

    u, v = pl.pallas_call(
        _uv_body,
        out_shape=(jax.ShapeDtypeStruct((n, h), jnp.float32),
                   jax.ShapeDtypeStruct((n, h), jnp.float32)),
    )(x, W_e1, b_e1.reshape(1, h))

    mesh = plsc.VectorSubcoreMesh(core_axis_name="c", subcore_axis_name="s",
                                  num_cores=NC, num_subcores=NS)
    epw = e // NW
    s = pl.kernel(
        _gather_add_body,
        out_type=jax.ShapeDtypeStruct((e, h), jnp.float32),
        mesh=mesh,
        scratch_types=[
            pltpu.VMEM((epw,), jnp.int32),
            pltpu.VMEM((epw,), jnp.int32),
            pltpu.VMEM((CH, h), jnp.float32),
            pltpu.VMEM((CH, h), jnp.float32),
            pltpu.VMEM((CH, h), jnp.float32),
            pltpu.VMEM((CH, h), jnp.float32),
            pltpu.VMEM((CH, h), jnp.float32),
            pltpu.VMEM((CH, h), jnp.float32),
            pltpu.SemaphoreType.DMA,
            pltpu.SemaphoreType.DMA,
            pltpu.SemaphoreType.DMA,
            pltpu.SemaphoreType.DMA,
        ],
    )(u, v, row, col)

    eb = 2000
    m = pl.pallas_call(
        _edge_body,
        grid=(e // eb,),
        in_specs=[
            pl.BlockSpec((eb, h), lambda i: (i, 0)),
            pl.BlockSpec((h, h), lambda i: (0, 0)),
            pl.BlockSpec((1, h), lambda i: (0, 0)),
        ],
        out_specs=pl.BlockSpec((eb, h), lambda i: (i, 0)),
        out_shape=jax.ShapeDtypeStruct((e, h), jnp.float32),
    )(s, W_e2, b_e2.reshape(1, h))

    zeros = jnp.zeros((n, h), jnp.float32)
    partials = pl.kernel(
        _scatter_add_body,
        out_type=jax.ShapeDtypeStruct((NC, n, h), jnp.float32),
        mesh=mesh,
        scratch_types=[
            pltpu.VMEM((CH,), jnp.int32),
            pltpu.VMEM((CH, h), jnp.float32),
            pltpu.VMEM((CH,), jnp.int32),
            pltpu.VMEM((CH, h), jnp.float32),
            pltpu.VMEM_SHARED((n, h), jnp.float32),
            pltpu.SemaphoreType.DMA,
            pltpu.SemaphoreType.DMA,
        ],
    )(m, row, zeros)

    nb = 2000
    hout = pl.pallas_call(
        _node_body,
        grid=(n // nb,),
        in_specs=[
            pl.BlockSpec((nb, d), lambda i: (i, 0)),
            pl.BlockSpec((NC, nb, h), lambda i: (0, i, 0)),
            pl.BlockSpec((h, d), lambda i: (0, 0)),
            pl.BlockSpec((h, h), lambda i: (0, 0)),
            pl.BlockSpec((1, h), lambda i: (0, 0)),
            pl.BlockSpec((d, h), lambda i: (0, 0)),
            pl.BlockSpec((1, d), lambda i: (0, 0)),
        ],
        out_specs=pl.BlockSpec((nb, d), lambda i: (i, 0)),
        out_shape=jax.ShapeDtypeStruct((n, d), jnp.float32),
    )(x, partials, W_n1[:, :d], W_n1[:, d:], b_n1.reshape(1, h),
      W_n2, b_n2.reshape(1, d))

    return (hout, m)
